# baseline (speedup 1.0000x reference)
import functools

import jax
import jax.numpy as jnp
from jax import lax
from jax.experimental import pallas as pl
from jax.experimental.pallas import tpu as pltpu
from jax.experimental.pallas import tpu_sc as plsc

NC, NS, L = 2, 16, 16
NW = NC * NS
EPS = 1e-7


def _rsqrt(x):
    i = lax.bitcast_convert_type(x, jnp.int32)
    i = jnp.int32(0x5F3759DF) - lax.shift_right_logical(i, 1)
    y = lax.bitcast_convert_type(i, jnp.float32)
    for _ in range(2):
        y = y * (1.5 - 0.5 * x * y * y)
    return y


def _make_kernel(B, S, V, D, P, affine):
    assert S % NW == 0 and D % L == 0
    SPW = S // NW
    CHUNK = 16
    HB = SPW // CHUNK
    DJ = D // L
    NBLK = B * HB
    RING = 4

    mesh = plsc.VectorSubcoreMesh(core_axis_name="c", subcore_axis_name="s")

    @functools.partial(
        pl.kernel,
        mesh=mesh,
        compiler_params=pltpu.CompilerParams(needs_layout_passes=False),
        out_type=jax.ShapeDtypeStruct((B, S, D), jnp.float32),
        scratch_types=[
            pltpu.VMEM((B, SPW), jnp.int32),
            pltpu.VMEM((2, CHUNK, D), jnp.float32),
            pltpu.VMEM((RING, CHUNK, D), jnp.float32),
            pltpu.VMEM((D,), jnp.float32),
            pltpu.VMEM((D,), jnp.float32),
            pltpu.VMEM((CHUNK, D // 2), jnp.int32),
            pltpu.SemaphoreType.DMA((RING,)),
            pltpu.SemaphoreType.DMA((RING,)),
            pltpu.SemaphoreType.DMA((2,)),
            pltpu.SemaphoreType.DMA,
        ],
    )
    def emb_kernel(ids_hbm, word_hbm, pos_hbm, gamma_hbm, beta_hbm, out_hbm,
                   idx_v, pos_v, rows_v, gam_v, bet_v, xb_v,
                   gsem, ssem, psem, isem):
        wid = lax.axis_index("s") * NC + lax.axis_index("c")
        s0 = wid * SPW

        pltpu.async_copy(pos_hbm.at[pl.ds(s0, CHUNK)], pos_v.at[0], psem.at[0])
        if affine:
            pltpu.sync_copy(gamma_hbm, gam_v)
            pltpu.sync_copy(beta_hbm, bet_v)
        for b in range(B):
            pltpu.async_copy(ids_hbm.at[b, pl.ds(s0, SPW)], idx_v.at[b], isem)
        for b in range(B):
            pltpu.make_async_copy(ids_hbm.at[0, pl.ds(0, SPW)], idx_v.at[b],
                                  isem).wait()

        inv_d = 1.0 / D

        NR = 8

        def compute_rows(buf, hp):
            JU = 8

            def row_quad(rr, _):
                rs = [rr * NR + n for n in range(NR)]

                def p1(jc, carry):
                    ss, qs = carry
                    ss, qs = list(ss), list(qs)
                    base = jc * (JU * L)
                    for jj in range(0, JU, 2):
                        d0 = pl.ds(base + jj * L, L)
                        d1 = pl.ds(base + (jj + 1) * L, L)
                        x0s = [rows_v[buf, r, d0] + pos_v[hp, r, d0] for r in rs]
                        x1s = [rows_v[buf, r, d1] + pos_v[hp, r, d1] for r in rs]
                        for n, r in enumerate(rs):
                            xb = plsc.pack(x0s[n], x1s[n],
                                           format=plsc.PackFormat.INTERLEAVED)
                            xb_v[r, pl.ds((base + jj * L) // 2, L)] = \
                                plsc.bitcast(xb, jnp.int32)
                            ss[n] = ss[n] + x0s[n] + x1s[n]
                            qs[n] = qs[n] + (x0s[n] * x0s[n] + x1s[n] * x1s[n])
                    return tuple(ss), tuple(qs)

                zeros = tuple(jnp.zeros((L,), jnp.float32) for _ in range(NR))
                ss, qs = lax.fori_loop(0, DJ // JU, p1, (zeros, zeros))
                means = [jnp.sum(s) * inv_d for s in ss]
                rstds = [_rsqrt(jnp.sum(q) * inv_d - m * m + EPS)
                         for q, m in zip(qs, means)]
                avs = [jnp.full((L,), r_, jnp.float32) for r_ in rstds]
                mbs = [jnp.full((L,), m * r_, jnp.float32)
                       for m, r_ in zip(means, rstds)]
                for j in range(0, DJ, 2):
                    d0 = pl.ds(j * L, L)
                    d1 = pl.ds((j + 1) * L, L)
                    for n, r in enumerate(rs):
                        xb = plsc.bitcast(xb_v[r, pl.ds(j * L // 2, L)],
                                          jnp.bfloat16)
                        x0, x1 = plsc.unpack(
                            xb, format=plsc.PackFormat.INTERLEAVED)
                        y0 = x0 * avs[n] - mbs[n]
                        y1 = x1 * avs[n] - mbs[n]
                        if affine:
                            y0 = y0 * gam_v[d0] + bet_v[d0]
                            y1 = y1 * gam_v[d1] + bet_v[d1]
                        rows_v[buf, r, d0] = y0
                        rows_v[buf, r, d1] = y1
                return 0
            lax.fori_loop(0, CHUNK // NR, row_quad, 0)

        def start_gather(k, buf):
            h, b = k // B, k % B
            pltpu.async_copy(word_hbm.at[idx_v.at[b, pl.ds(h * CHUNK, CHUNK)]],
                             rows_v.at[buf], gsem.at[buf])

        def wait_gather(buf):
            pltpu.make_async_copy(word_hbm.at[pl.ds(0, CHUNK)], rows_v.at[buf],
                                  gsem.at[buf]).wait()

        def start_store(k, buf):
            h, b = k // B, k % B
            pltpu.async_copy(rows_v.at[buf],
                             out_hbm.at[b, pl.ds(s0 + h * CHUNK, CHUNK)],
                             ssem.at[buf])

        def wait_store(buf):
            pltpu.make_async_copy(rows_v.at[buf],
                                  out_hbm.at[0, pl.ds(0, CHUNK)],
                                  ssem.at[buf]).wait()

        def wait_pos(hp):
            pltpu.make_async_copy(pos_hbm.at[pl.ds(0, CHUNK)], pos_v.at[hp],
                                  psem.at[hp]).wait()

        start_gather(0, 0)
        start_gather(1, 1)

        def step(k, _):
            buf = lax.rem(k, RING)
            h, b = k // B, k % B
            ahead = lax.rem(k + 2, RING)

            @pl.when(k + 2 < NBLK)
            def _():
                @pl.when(k >= 2)
                def _():
                    wait_store(ahead)
                start_gather(k + 2, ahead)

            @pl.when(b == 0)
            def _():
                hp = lax.rem(h, 2)
                wait_pos(hp)

                @pl.when(h + 1 < HB)
                def _():
                    pltpu.async_copy(
                        pos_hbm.at[pl.ds(s0 + (h + 1) * CHUNK, CHUNK)],
                        pos_v.at[1 - hp], psem.at[1 - hp])

            wait_gather(buf)
            compute_rows(buf, lax.rem(h, 2))
            start_store(k, buf)
            return 0

        lax.fori_loop(0, NBLK, step, 0)
        for buf in range(RING):
            wait_store(buf)

    return emb_kernel


def kernel(input_ids, word_emb, pos_emb, gamma, beta):
    B, S = input_ids.shape
    V, D = word_emb.shape
    P = pos_emb.shape[0]
    ids = input_ids.astype(jnp.int32)
    fast = _make_kernel(B, S, V, D, P, affine=False)
    general = _make_kernel(B, S, V, D, P, affine=True)
    identity = jnp.logical_and(jnp.all(gamma == 1.0), jnp.all(beta == 0.0))
    return lax.cond(
        identity,
        lambda operands: fast(*operands),
        lambda operands: general(*operands),
        (ids, word_emb, pos_emb, gamma, beta),
    )


# speedup vs baseline: 2.6219x; 2.6219x over previous
import functools

import jax
import jax.numpy as jnp
from jax import lax
from jax.experimental import pallas as pl
from jax.experimental.pallas import tpu as pltpu
from jax.experimental.pallas import tpu_sc as plsc

NC, NS, L = 2, 16, 16
NW = NC * NS
EPS = 1e-7


def _rsqrt(x):
    i = lax.bitcast_convert_type(x, jnp.int32)
    i = jnp.int32(0x5F3759DF) - lax.shift_right_logical(i, 1)
    y = lax.bitcast_convert_type(i, jnp.float32)
    for _ in range(2):
        y = y * (1.5 - 0.5 * x * y * y)
    return y


def _make_kernel(B, S, V, D, P, affine):
    assert S % NW == 0 and D % L == 0
    SPW = S // NW
    CHUNK = 16
    HB = SPW // CHUNK
    DJ = D // L
    NBLK = B * HB
    RING = 4

    mesh = plsc.VectorSubcoreMesh(core_axis_name="c", subcore_axis_name="s")

    @functools.partial(
        pl.kernel,
        mesh=mesh,
        compiler_params=pltpu.CompilerParams(needs_layout_passes=False),
        out_type=jax.ShapeDtypeStruct((B, S, D), jnp.float32),
        scratch_types=[
            pltpu.VMEM((B, SPW), jnp.int32),
            pltpu.VMEM((2, CHUNK, D), jnp.float32),
            pltpu.VMEM((RING, CHUNK, D), jnp.float32),
            pltpu.VMEM((D,), jnp.float32),
            pltpu.VMEM((D,), jnp.float32),
            pltpu.SemaphoreType.DMA((RING,)),
            pltpu.SemaphoreType.DMA((RING,)),
            pltpu.SemaphoreType.DMA((2,)),
            pltpu.SemaphoreType.DMA,
        ],
    )
    def emb_kernel(ids_hbm, word_hbm, pos_hbm, gamma_hbm, beta_hbm, out_hbm,
                   idx_v, pos_v, rows_v, gam_v, bet_v, gsem, ssem, psem, isem):
        wid = lax.axis_index("s") * NC + lax.axis_index("c")
        s0 = wid * SPW

        pltpu.async_copy(pos_hbm.at[pl.ds(s0, CHUNK)], pos_v.at[0], psem.at[0])
        if affine:
            pltpu.sync_copy(gamma_hbm, gam_v)
            pltpu.sync_copy(beta_hbm, bet_v)
        for b in range(B):
            pltpu.async_copy(ids_hbm.at[b, pl.ds(s0, SPW)], idx_v.at[b], isem)
        for b in range(B):
            pltpu.make_async_copy(ids_hbm.at[0, pl.ds(0, SPW)], idx_v.at[b],
                                  isem).wait()

        inv_d = 1.0 / D

        NR = 8

        def compute_rows(buf, hp):
            JU = 8

            def row_quad(rr, _):
                rs = [rr * NR + n for n in range(NR)]

                def p1(jc, carry):
                    ss, qs = carry
                    ss, qs = list(ss), list(qs)
                    base = jc * (JU * L)
                    for jj in range(JU):
                        d = pl.ds(base + jj * L, L)
                        xs = [rows_v[buf, r, d] + pos_v[hp, r, d] for r in rs]
                        for n, r in enumerate(rs):
                            rows_v[buf, r, d] = xs[n]
                            ss[n] = ss[n] + xs[n]
                            qs[n] = qs[n] + xs[n] * xs[n]
                    return tuple(ss), tuple(qs)

                zeros = tuple(jnp.zeros((L,), jnp.float32) for _ in range(NR))
                ss, qs = lax.fori_loop(0, DJ // JU, p1, (zeros, zeros))
                means = [jnp.sum(s) * inv_d for s in ss]
                rstds = [_rsqrt(jnp.sum(q) * inv_d - m * m + EPS)
                         for q, m in zip(qs, means)]
                avs = [jnp.full((L,), r_, jnp.float32) for r_ in rstds]
                mbs = [jnp.full((L,), m * r_, jnp.float32)
                       for m, r_ in zip(means, rstds)]
                for j in range(DJ):
                    d = pl.ds(j * L, L)
                    for n, r in enumerate(rs):
                        y = rows_v[buf, r, d] * avs[n] - mbs[n]
                        if affine:
                            y = y * gam_v[d] + bet_v[d]
                        rows_v[buf, r, d] = y
                return 0
            lax.fori_loop(0, CHUNK // NR, row_quad, 0)

        def start_gather(k, buf):
            h, b = k // B, k % B
            pltpu.async_copy(word_hbm.at[idx_v.at[b, pl.ds(h * CHUNK, CHUNK)]],
                             rows_v.at[buf], gsem.at[buf])

        def wait_gather(buf):
            pltpu.make_async_copy(word_hbm.at[pl.ds(0, CHUNK)], rows_v.at[buf],
                                  gsem.at[buf]).wait()

        def start_store(k, buf):
            h, b = k // B, k % B
            pltpu.async_copy(rows_v.at[buf],
                             out_hbm.at[b, pl.ds(s0 + h * CHUNK, CHUNK)],
                             ssem.at[buf])

        def wait_store(buf):
            pltpu.make_async_copy(rows_v.at[buf],
                                  out_hbm.at[0, pl.ds(0, CHUNK)],
                                  ssem.at[buf]).wait()

        def wait_pos(hp):
            pltpu.make_async_copy(pos_hbm.at[pl.ds(0, CHUNK)], pos_v.at[hp],
                                  psem.at[hp]).wait()

        start_gather(0, 0)
        start_gather(1, 1)

        def step(k, _):
            buf = lax.rem(k, RING)
            h, b = k // B, k % B
            ahead = lax.rem(k + 2, RING)

            @pl.when(k + 2 < NBLK)
            def _():
                @pl.when(k >= 2)
                def _():
                    wait_store(ahead)
                start_gather(k + 2, ahead)

            @pl.when(b == 0)
            def _():
                hp = lax.rem(h, 2)
                wait_pos(hp)

                @pl.when(h + 1 < HB)
                def _():
                    pltpu.async_copy(
                        pos_hbm.at[pl.ds(s0 + (h + 1) * CHUNK, CHUNK)],
                        pos_v.at[1 - hp], psem.at[1 - hp])

            wait_gather(buf)
            compute_rows(buf, lax.rem(h, 2))
            start_store(k, buf)
            return 0

        lax.fori_loop(0, NBLK, step, 0)
        for buf in range(RING):
            wait_store(buf)

    return emb_kernel


def kernel(input_ids, word_emb, pos_emb, gamma, beta):
    B, S = input_ids.shape
    V, D = word_emb.shape
    P = pos_emb.shape[0]
    ids = input_ids.astype(jnp.int32)
    fast = _make_kernel(B, S, V, D, P, affine=False)
    general = _make_kernel(B, S, V, D, P, affine=True)
    identity = jnp.logical_and(jnp.all(gamma == 1.0), jnp.all(beta == 0.0))
    return lax.cond(
        identity,
        lambda operands: fast(*operands),
        lambda operands: general(*operands),
        (ids, word_emb, pos_emb, gamma, beta),
    )
